# Initial kernel scaffold; baseline (speedup 1.0000x reference)
import jax
import jax.numpy as jnp
from jax.experimental import pallas as pl


def kernel(inputs, lookup):
    raise NotImplementedError("write your pallas kernel here")


# speedup vs baseline: 288.8898x; 288.8898x over previous
import functools

import jax
import jax.numpy as jnp
from jax import lax
from jax.experimental import pallas as pl
from jax.experimental.pallas import tpu as pltpu
from jax.experimental.pallas import tpu_sc as plsc

_NC = 2
_NS = 16
_NW = _NC * _NS
_L = 16

_TOTAL = 16384 * 200
_PER_W = _TOTAL // _NW
_CHUNK = 12800
_NCHUNK = _PER_W // _CHUNK
_NBUF = 2


def _sc_body(in_hbm, lut_hbm, out_hbm, table_v, in_v, out_v, in_sems, out_sems):
    wid = lax.axis_index("s") * _NC + lax.axis_index("c")
    base = wid * _PER_W

    pltpu.sync_copy(lut_hbm, table_v)

    for b in range(_NBUF):
        pltpu.async_copy(
            in_hbm.at[pl.ds(base + b * _CHUNK, _CHUNK)], in_v.at[b], in_sems.at[b]
        )

    for c in range(_NCHUNK):
        b = c % _NBUF
        pltpu.make_async_copy(
            in_hbm.at[pl.ds(base + c * _CHUNK, _CHUNK)], in_v.at[b], in_sems.at[b]
        ).wait()
        if c >= _NBUF:
            pltpu.make_async_copy(
                out_v.at[b],
                out_hbm.at[pl.ds(base + (c - _NBUF) * _CHUNK, _CHUNK)],
                out_sems.at[b],
            ).wait()

        @plsc.parallel_loop(0, _CHUNK, _L, unroll=8)
        def _gather_chunk(i):
            x = in_v[b, pl.ds(i, _L)]
            idx = jnp.clip((x * 255.0).astype(jnp.int32), 0, 255)
            out_v[b, pl.ds(i, _L)] = plsc.load_gather(table_v, [idx])

        pltpu.async_copy(
            out_v.at[b], out_hbm.at[pl.ds(base + c * _CHUNK, _CHUNK)], out_sems.at[b]
        )
        if c + _NBUF < _NCHUNK:
            pltpu.async_copy(
                in_hbm.at[pl.ds(base + (c + _NBUF) * _CHUNK, _CHUNK)],
                in_v.at[b],
                in_sems.at[b],
            )

    for c in range(_NCHUNK - _NBUF, _NCHUNK):
        b = c % _NBUF
        pltpu.make_async_copy(
            out_v.at[b], out_hbm.at[pl.ds(base + c * _CHUNK, _CHUNK)], out_sems.at[b]
        ).wait()


@jax.jit
def _run(flat_inputs, lookup):
    mesh = plsc.VectorSubcoreMesh(core_axis_name="c", subcore_axis_name="s")
    return pl.kernel(
        _sc_body,
        out_type=jax.ShapeDtypeStruct((_TOTAL,), jnp.float32),
        mesh=mesh,
        scratch_types=[
            pltpu.VMEM((256,), jnp.float32),
            pltpu.VMEM((_NBUF, _CHUNK), jnp.float32),
            pltpu.VMEM((_NBUF, _CHUNK), jnp.float32),
            pltpu.SemaphoreType.DMA((_NBUF,)),
            pltpu.SemaphoreType.DMA((_NBUF,)),
        ],
        compiler_params=pltpu.CompilerParams(needs_layout_passes=False),
    )(flat_inputs, lookup)


def kernel(inputs, lookup):
    flat = jnp.reshape(inputs, (_TOTAL,))
    out = _run(flat, lookup)
    return jnp.reshape(out, inputs.shape)


# speedup vs baseline: 523.5642x; 1.8123x over previous
import functools

import jax
import jax.numpy as jnp
from jax import lax
from jax.experimental import pallas as pl
from jax.experimental.pallas import tpu as pltpu
from jax.experimental.pallas import tpu_sc as plsc

_NC = 2
_NS = 16
_NW = _NC * _NS
_L = 16

_ROWS = 16384
_COLS = 200
_TOTAL = _ROWS * _COLS
_ROWS_W = _ROWS // _NW
_CROWS = 64
_CHUNK = _CROWS * _COLS
_NCHUNK = _ROWS_W // _CROWS
_NBUF = 2


def _sc_body(in_hbm, lut_hbm, out_hbm, table_v, in_v, out_v, in_sems, out_sems):
    wid = lax.axis_index("s") * _NC + lax.axis_index("c")
    base_row = wid * _ROWS_W

    pltpu.sync_copy(lut_hbm, table_v)

    for b in range(_NBUF):
        pltpu.async_copy(
            in_hbm.at[pl.ds(base_row + b * _CROWS, _CROWS), :],
            in_v.at[b],
            in_sems.at[b],
        )

    for c in range(_NCHUNK):
        b = c % _NBUF
        pltpu.make_async_copy(
            in_hbm.at[pl.ds(base_row + c * _CROWS, _CROWS), :],
            in_v.at[b],
            in_sems.at[b],
        ).wait()
        if c >= _NBUF:
            pltpu.make_async_copy(
                out_v.at[b],
                out_hbm.at[pl.ds(base_row + (c - _NBUF) * _CROWS, _CROWS), :],
                out_sems.at[b],
            ).wait()

        col_offs = [k * _L for k in range(_COLS // _L)] + [_COLS - _L]

        @plsc.parallel_loop(0, _CROWS, 1, unroll=2)
        def _gather_rows(r):
            for off in col_offs:
                x = in_v[b, r, pl.ds(off, _L)]
                idx = jnp.clip((x * 255.0).astype(jnp.int32), 0, 255)
                out_v[b, r, pl.ds(off, _L)] = plsc.load_gather(table_v, [idx])

        pltpu.async_copy(
            out_v.at[b],
            out_hbm.at[pl.ds(base_row + c * _CROWS, _CROWS), :],
            out_sems.at[b],
        )
        if c + _NBUF < _NCHUNK:
            pltpu.async_copy(
                in_hbm.at[pl.ds(base_row + (c + _NBUF) * _CROWS, _CROWS), :],
                in_v.at[b],
                in_sems.at[b],
            )

    for c in range(_NCHUNK - _NBUF, _NCHUNK):
        b = c % _NBUF
        pltpu.make_async_copy(
            out_v.at[b],
            out_hbm.at[pl.ds(base_row + c * _CROWS, _CROWS), :],
            out_sems.at[b],
        ).wait()


@jax.jit
def _run(inputs, lookup):
    mesh = plsc.VectorSubcoreMesh(core_axis_name="c", subcore_axis_name="s")
    return pl.kernel(
        _sc_body,
        out_type=jax.ShapeDtypeStruct((_ROWS, _COLS), jnp.float32),
        mesh=mesh,
        scratch_types=[
            pltpu.VMEM((256,), jnp.float32),
            pltpu.VMEM((_NBUF, _CROWS, _COLS), jnp.float32),
            pltpu.VMEM((_NBUF, _CROWS, _COLS), jnp.float32),
            pltpu.SemaphoreType.DMA((_NBUF,)),
            pltpu.SemaphoreType.DMA((_NBUF,)),
        ],
        compiler_params=pltpu.CompilerParams(needs_layout_passes=False),
    )(inputs, lookup)


def kernel(inputs, lookup):
    return _run(inputs, lookup)
